# baseline (speedup 1.0000x reference)
import functools

import jax
import jax.numpy as jnp
from jax import lax
from jax.experimental import pallas as pl
from jax.experimental.pallas import tpu as pltpu
from jax.experimental.pallas import tpu_sc as plsc

_CH = 128


def _sc_gather(node_i, node_j, neg_flat, table):
    info = plsc.get_sparse_core_info()
    nc, ns = info.num_cores, info.num_subcores
    nw = nc * ns
    b = node_i.shape[0]
    d = table.shape[1]
    nneg = neg_flat.shape[0]
    rpw_b = b // nw
    rpw_n = nneg // nw
    mesh = plsc.VectorSubcoreMesh(core_axis_name="c", subcore_axis_name="s")

    @functools.partial(
        pl.kernel,
        mesh=mesh,
        out_type=[
            jax.ShapeDtypeStruct((b, d), jnp.float32),
            jax.ShapeDtypeStruct((b, d), jnp.float32),
            jax.ShapeDtypeStruct((nneg, d), jnp.float32),
        ],
        scratch_types=[
            pltpu.VMEM((_CH,), jnp.int32),
            pltpu.VMEM((_CH, d), jnp.float32),
            pltpu.SemaphoreType.DMA,
        ],
        compiler_params=pltpu.CompilerParams(use_tc_tiling_on_sc=False),
    )
    def sc_fn(ni, nj, nn, tbl, out_i, out_j, out_n, idx_v, rows_v, sem):
        wid = lax.axis_index("s") * nc + lax.axis_index("c")

        def run(idx_hbm, out_hbm, base, nch):
            for t in range(nch):
                off = base + t * _CH
                pltpu.sync_copy(idx_hbm.at[pl.ds(off, _CH)], idx_v)
                pltpu.async_copy(tbl.at[idx_v], rows_v, sem).wait()
                pltpu.sync_copy(rows_v, out_hbm.at[pl.ds(off, _CH)])

        run(ni, out_i, wid * rpw_b, rpw_b // _CH)
        run(nj, out_j, wid * rpw_b, rpw_b // _CH)
        run(nn, out_n, wid * rpw_n, rpw_n // _CH)

    return sc_fn(node_i, node_j, neg_flat, table)


def _tc_loss(ei, ej, en):
    b, d = ei.shape
    neg = en.shape[1] // d
    bb = 8192
    r = 1024

    def body(ei_ref, ej_ref, en_ref, out_ref):
        a = ei_ref[...]
        pos = jnp.sum(a * ej_ref[...], axis=-1)
        loss = -jnp.log(jax.nn.sigmoid(pos))
        for k in range(neg):
            s = jnp.sum(a * en_ref[:, k * d:(k + 1) * d], axis=-1)
            loss = loss - jnp.log(1.0 - jax.nn.sigmoid(s))
        out_ref[...] = loss.reshape(bb // r, r)

    out = pl.pallas_call(
        body,
        grid=(b // bb,),
        in_specs=[
            pl.BlockSpec((bb, d), lambda i: (i, 0)),
            pl.BlockSpec((bb, d), lambda i: (i, 0)),
            pl.BlockSpec((bb, neg * d), lambda i: (i, 0)),
        ],
        out_specs=pl.BlockSpec((bb // r, r), lambda i: (i, 0)),
        out_shape=jax.ShapeDtypeStruct((b // r, r), jnp.float32),
    )(ei, ej, en)
    return out.reshape(b)


def kernel(node_i, node_j, neg_samples, table):
    b, neg = neg_samples.shape
    d = table.shape[1]
    ni = node_i.astype(jnp.int32)
    nj = node_j.astype(jnp.int32)
    nn = neg_samples.reshape(-1).astype(jnp.int32)
    gi, gj, gn = _sc_gather(ni, nj, nn, table)
    return _tc_loss(gi, gj, gn.reshape(b, neg * d))
